# baseline (speedup 1.0000x reference)
import jax
import jax.numpy as jnp
from jax import lax
from jax.experimental import pallas as pl
from jax.experimental.pallas import tpu as pltpu

B, K, H, W = 32, 17, 128, 128


def _fk(k):
    return jnp.where(k == 0, 0, jnp.where(k % 2 == 1, k + 1, k - 1))


def _rev_idx(shape, dim):
    return (W - 1) - lax.broadcasted_iota(jnp.int32, shape, dim)


def _flip4(x):
    def body(in_ref, out_ref):
        v = in_ref[0, 0]
        out_ref[0, 0] = jnp.take_along_axis(v, _rev_idx(v.shape, 1), axis=1)

    return pl.pallas_call(
        body,
        grid=(B, K),
        in_specs=[pl.BlockSpec((1, 1, H, W), lambda b, k: (b, _fk(k), 0, 0))],
        out_specs=pl.BlockSpec((1, 1, H, W), lambda b, k: (b, k, 0, 0)),
        out_shape=jax.ShapeDtypeStruct((B, K, H, W), jnp.float32),
    )(x)


def _flip5(x):
    def body(in_ref, out_ref):
        v = in_ref[0, 0, 0]
        v = jnp.take_along_axis(v, _rev_idx(v.shape, 1), axis=1)
        c = pl.program_id(2)
        out_ref[0, 0, 0] = jnp.where(c == 0, -v, v)

    return pl.pallas_call(
        body,
        grid=(B, K, 2),
        in_specs=[pl.BlockSpec((1, 1, 1, H, W),
                               lambda b, k, c: (b, _fk(k), c, 0, 0))],
        out_specs=pl.BlockSpec((1, 1, 1, H, W),
                               lambda b, k, c: (b, k, c, 0, 0)),
        out_shape=jax.ShapeDtypeStruct((B, K, 2, H, W), jnp.float32),
    )(x)


def kernel(field_conf, field_reg, field_scale):
    return (_flip4(field_conf), _flip5(field_reg), _flip4(field_scale))


# speedup vs baseline: 2.8266x; 2.8266x over previous
import jax
import jax.numpy as jnp
from jax import lax
from jax.experimental import pallas as pl
from jax.experimental.pallas import tpu as pltpu

B, K, H, W = 32, 17, 128, 128


def _fk(k):
    return jnp.where(k == 0, 0, jnp.where(k % 2 == 1, k + 1, k - 1))


def _antidiag():
    r = lax.broadcasted_iota(jnp.int32, (W, W), 0)
    c = lax.broadcasted_iota(jnp.int32, (W, W), 1)
    return jnp.where(r + c == W - 1, 1.0, 0.0).astype(jnp.float32)


def _flip4(x, bb):
    def body(in_ref, out_ref):
        j = _antidiag()
        for i in range(bb):
            out_ref[i, 0] = jax.lax.dot(
                in_ref[i, 0], j,
                precision=jax.lax.Precision.HIGHEST,
                preferred_element_type=jnp.float32)

    return pl.pallas_call(
        body,
        grid=(B // bb, K),
        in_specs=[pl.BlockSpec((bb, 1, H, W), lambda b, k: (b, _fk(k), 0, 0))],
        out_specs=pl.BlockSpec((bb, 1, H, W), lambda b, k: (b, k, 0, 0)),
        out_shape=jax.ShapeDtypeStruct((B, K, H, W), jnp.float32),
    )(x)


def _flip5(x, bb):
    def body(in_ref, out_ref):
        j = _antidiag()
        c = pl.program_id(2)
        sign = jnp.where(c == 0, -1.0, 1.0)
        for i in range(bb):
            out_ref[i, 0, 0] = sign * jax.lax.dot(
                in_ref[i, 0, 0], j,
                precision=jax.lax.Precision.HIGHEST,
                preferred_element_type=jnp.float32)

    return pl.pallas_call(
        body,
        grid=(B // bb, K, 2),
        in_specs=[pl.BlockSpec((bb, 1, 1, H, W),
                               lambda b, k, c: (b, _fk(k), c, 0, 0))],
        out_specs=pl.BlockSpec((bb, 1, 1, H, W),
                               lambda b, k, c: (b, k, c, 0, 0)),
        out_shape=jax.ShapeDtypeStruct((B, K, 2, H, W), jnp.float32),
    )(x)


def kernel(field_conf, field_reg, field_scale):
    return (_flip4(field_conf, 4), _flip5(field_reg, 4), _flip4(field_scale, 4))


# speedup vs baseline: 7.5027x; 2.6543x over previous
import functools

import jax
import jax.numpy as jnp
from jax import lax
from jax.experimental import pallas as pl
from jax.experimental.pallas import tpu as pltpu
from jax.experimental.pallas import tpu_sc as plsc

B, K, H, W = 32, 17, 128, 128
HW = H * W
_FI = tuple(0 if k == 0 else (k + 1 if k % 2 == 1 else k - 1) for k in range(K))

_CHUNKS_PER_ROW = W // 16


def _rev_plane(vin, vout, negate):
    def body(h, carry):
        base = h * W
        for j in range(_CHUNKS_PER_ROW):
            src = base + (_CHUNKS_PER_ROW - 1 - j) * 16
            v = lax.rev(vin[pl.ds(src, 16)], (0,))
            if negate:
                v = -v
            vout[pl.ds(base + j * 16, 16)] = v
        return carry

    lax.fori_loop(0, H, body, 0)


def _sc_flip(conf, reg):
    mesh = plsc.VectorSubcoreMesh(core_axis_name="c", subcore_axis_name="s")

    @functools.partial(
        pl.kernel,
        mesh=mesh,
        out_type=(
            jax.ShapeDtypeStruct((B, K, HW), jnp.float32),
            jax.ShapeDtypeStruct((B, K, 2, HW), jnp.float32),
        ),
        scratch_types=[
            pltpu.VMEM((HW,), jnp.float32),
            pltpu.VMEM((HW,), jnp.float32),
            pltpu.VMEM((HW,), jnp.float32),
            pltpu.VMEM((HW,), jnp.float32),
            pltpu.SemaphoreType.DMA,
            pltpu.SemaphoreType.DMA,
            pltpu.SemaphoreType.DMA,
            pltpu.SemaphoreType.DMA,
        ],
        compiler_params=pltpu.CompilerParams(use_tc_tiling_on_sc=False),
    )
    def k(conf_in, reg_in, conf_out, reg_out,
          bin0, bin1, bout0, bout1, isem0, isem1, osem0, osem1):
        w = lax.axis_index("s") * 2 + lax.axis_index("c")
        bins, bouts = (bin0, bin1), (bout0, bout1)
        isems, osems = (isem0, isem1), (osem0, osem1)

        planes = []
        for kk in range(K):
            planes.append((conf_in.at[w, _FI[kk]], conf_out.at[w, kk], False))
        for c in range(2):
            for kk in range(K):
                planes.append(
                    (reg_in.at[w, _FI[kk], c], reg_out.at[w, kk, c], c == 0))
        n = len(planes)

        copy_in = [None] * n
        copy_out = [None] * n
        copy_in[0] = pltpu.async_copy(planes[0][0], bins[0], isems[0])
        copy_in[1] = pltpu.async_copy(planes[1][0], bins[1], isems[1])
        for i in range(n):
            s = i % 2
            copy_in[i].wait()
            if i >= 2:
                copy_out[i - 2].wait()
            _rev_plane(bins[s], bouts[s], planes[i][2])
            copy_out[i] = pltpu.async_copy(bouts[s], planes[i][1], osems[s])
            if i + 2 < n:
                copy_in[i + 2] = pltpu.async_copy(
                    planes[i + 2][0], bins[s], isems[s])
        copy_out[n - 2].wait()
        copy_out[n - 1].wait()

    return k(conf, reg)


def _fk(k):
    return jnp.where(k == 0, 0, jnp.where(k % 2 == 1, k + 1, k - 1))


def _tc_flip(x, bb=4):
    def body(in_ref, out_ref):
        r = lax.broadcasted_iota(jnp.int32, (W, W), 0)
        c = lax.broadcasted_iota(jnp.int32, (W, W), 1)
        j = jnp.where(r + c == W - 1, 1.0, 0.0).astype(jnp.float32)
        for i in range(bb):
            out_ref[i, 0] = jax.lax.dot(
                in_ref[i, 0], j,
                precision=jax.lax.Precision.HIGHEST,
                preferred_element_type=jnp.float32)

    return pl.pallas_call(
        body,
        grid=(B // bb, K),
        in_specs=[pl.BlockSpec((bb, 1, H, W), lambda b, k: (b, _fk(k), 0, 0))],
        out_specs=pl.BlockSpec((bb, 1, H, W), lambda b, k: (b, k, 0, 0)),
        out_shape=jax.ShapeDtypeStruct((B, K, H, W), jnp.float32),
    )(x)


def kernel(field_conf, field_reg, field_scale):
    conf = field_conf.reshape(B, K, HW)
    reg = field_reg.reshape(B, K, 2, HW)
    oc, orr = _sc_flip(conf, reg)
    osc = _tc_flip(field_scale)
    return (
        oc.reshape(B, K, H, W),
        orr.reshape(B, K, 2, H, W),
        osc,
    )


# speedup vs baseline: 9.4282x; 1.2566x over previous
import functools

import jax
import jax.numpy as jnp
from jax import lax
from jax.experimental import pallas as pl
from jax.experimental.pallas import tpu as pltpu
from jax.experimental.pallas import tpu_sc as plsc

B, K, H, W = 32, 17, 128, 128
HW = H * W
_FI = tuple(0 if k == 0 else (k + 1 if k % 2 == 1 else k - 1) for k in range(K))

_CHUNKS_PER_ROW = W // 16


def _rev_plane(vin, vout, negate):
    def body(h, carry):
        base = h * W
        for j in range(_CHUNKS_PER_ROW):
            src = base + (_CHUNKS_PER_ROW - 1 - j) * 16
            v = lax.rev(vin[pl.ds(src, 16)], (0,))
            if negate:
                v = -v
            vout[pl.ds(base + j * 16, 16)] = v
        return carry

    lax.fori_loop(0, H, body, 0)


def _sc_flip(conf, reg, scale):
    mesh = plsc.VectorSubcoreMesh(core_axis_name="c", subcore_axis_name="s")

    @functools.partial(
        pl.kernel,
        mesh=mesh,
        out_type=(
            jax.ShapeDtypeStruct((B, K, HW), jnp.float32),
            jax.ShapeDtypeStruct((B, K, 2, HW), jnp.float32),
            jax.ShapeDtypeStruct((B, K, HW), jnp.float32),
        ),
        scratch_types=[
            pltpu.VMEM((HW,), jnp.float32),
            pltpu.VMEM((HW,), jnp.float32),
            pltpu.VMEM((HW,), jnp.float32),
            pltpu.VMEM((HW,), jnp.float32),
            pltpu.VMEM((HW,), jnp.float32),
            pltpu.VMEM((HW,), jnp.float32),
            pltpu.SemaphoreType.DMA,
            pltpu.SemaphoreType.DMA,
            pltpu.SemaphoreType.DMA,
            pltpu.SemaphoreType.DMA,
            pltpu.SemaphoreType.DMA,
            pltpu.SemaphoreType.DMA,
        ],
        compiler_params=pltpu.CompilerParams(use_tc_tiling_on_sc=False),
    )
    def k(conf_in, reg_in, scale_in, conf_out, reg_out, scale_out,
          bin0, bin1, bin2, bout0, bout1, bout2,
          isem0, isem1, isem2, osem0, osem1, osem2):
        w = lax.axis_index("s") * 2 + lax.axis_index("c")
        bins, bouts = (bin0, bin1, bin2), (bout0, bout1, bout2)
        isems, osems = (isem0, isem1, isem2), (osem0, osem1, osem2)

        planes = []
        for src_ref, dst_ref in ((conf_in, conf_out), (scale_in, scale_out)):
            for kk in range(K):
                planes.append((src_ref.at[w, _FI[kk]], dst_ref.at[w, kk], False))
        for c in range(2):
            for kk in range(K):
                planes.append(
                    (reg_in.at[w, _FI[kk], c], reg_out.at[w, kk, c], c == 0))
        n = len(planes)

        d = 3
        copy_in = [None] * n
        copy_out = [None] * n
        for i in range(d):
            copy_in[i] = pltpu.async_copy(planes[i][0], bins[i], isems[i])
        for i in range(n):
            s = i % d
            copy_in[i].wait()
            if i >= d:
                copy_out[i - d].wait()
            _rev_plane(bins[s], bouts[s], planes[i][2])
            copy_out[i] = pltpu.async_copy(bouts[s], planes[i][1], osems[s])
            if i + d < n:
                copy_in[i + d] = pltpu.async_copy(
                    planes[i + d][0], bins[s], isems[s])
        for i in range(n - d, n):
            copy_out[i].wait()

    return k(conf, reg, scale)


def kernel(field_conf, field_reg, field_scale):
    conf = field_conf.reshape(B, K, HW)
    reg = field_reg.reshape(B, K, 2, HW)
    scale = field_scale.reshape(B, K, HW)
    oc, orr, os = _sc_flip(conf, reg, scale)
    return (
        oc.reshape(B, K, H, W),
        orr.reshape(B, K, 2, H, W),
        os.reshape(B, K, H, W),
    )
